# Initial kernel scaffold; baseline (speedup 1.0000x reference)
import jax
import jax.numpy as jnp
from jax.experimental import pallas as pl


def kernel(inputs, edge_index, edge_attr, W1, a_src1, a_dst1, b1, W2, a_src2, a_dst2, b2):
    raise NotImplementedError("write your pallas kernel here")


# speedup vs baseline: 38.9180x; 38.9180x over previous
import functools

import jax
import jax.numpy as jnp
from jax import lax
from jax.experimental import pallas as pl
from jax.experimental.pallas import tpu as pltpu
from jax.experimental.pallas import tpu_sc as plsc

N = 10000
F_IN = 128
H1 = 8
C1 = 16
HC = H1 * C1
NCLS = 32

NPAD = 10240
NC = 2
NS = 16
NW = NC * NS
CH = 128
RPT = NPAD // NS
BN = 1024

_mesh = plsc.VectorSubcoreMesh(
    core_axis_name="c", subcore_axis_name="s", num_cores=NC, num_subcores=NS)


def _k_from_mk(mkbuf):
    return mkbuf[0] + mkbuf[1]


def _splat(v, h):
    idx = jnp.full((16, 1), h, jnp.int32)
    dnums = lax.GatherDimensionNumbers(
        offset_dims=(), collapsed_slice_dims=(0,), start_index_map=(0,))
    return lax.gather(v, idx, dnums, (1,),
                      mode=lax.GatherScatterMode.PROMISE_IN_BOUNDS)


def _prep1_body(x_ref, w_ref, aws_ref, awd_ref, ht_ref, as_ref, ad_ref, mk_ref):
    i = pl.program_id(0)
    h = jnp.dot(x_ref[...], w_ref[...], preferred_element_type=jnp.float32)
    ht_ref[...] = h
    a_s = jnp.dot(h, aws_ref[...], preferred_element_type=jnp.float32)
    a_d = jnp.dot(h, awd_ref[...], preferred_element_type=jnp.float32)
    as_ref[...] = a_s
    ad_ref[...] = a_d
    bm = jnp.concatenate([jnp.full((1, 16), jnp.max(a_s), jnp.float32),
                          jnp.full((1, 16), jnp.max(a_d), jnp.float32)],
                         axis=0)

    @pl.when(i == 0)
    def _():
        mk_ref[...] = bm

    @pl.when(i > 0)
    def _():
        mk_ref[...] = jnp.maximum(mk_ref[...], bm)


def _prep2_body(o0_ref, o1_ref, b_ref, w_ref, aws_ref, awd_ref,
                ht_ref, as_ref, ad_ref, mk_ref):
    i = pl.program_id(0)
    x2 = jax.nn.relu(o0_ref[...] + o1_ref[...] + b_ref[...])
    h = jnp.dot(x2, w_ref[...], preferred_element_type=jnp.float32)
    ht_ref[...] = h
    a_s = jnp.dot(h, aws_ref[...], preferred_element_type=jnp.float32)
    a_d = jnp.dot(h, awd_ref[...], preferred_element_type=jnp.float32)
    as_ref[...] = a_s
    ad_ref[...] = a_d
    bm = jnp.concatenate([jnp.full((1, 16), jnp.max(a_s), jnp.float32),
                          jnp.full((1, 16), jnp.max(a_d), jnp.float32)],
                         axis=0)

    @pl.when(i == 0)
    def _():
        mk_ref[...] = bm

    @pl.when(i > 0)
    def _():
        mk_ref[...] = jnp.maximum(mk_ref[...], bm)


def _final_body(o0_ref, o1_ref, b_ref, out_ref):
    out_ref[...] = o0_ref[...] + o1_ref[...] + b_ref[...]


def _make_prep1():
    return pl.pallas_call(
        _prep1_body,
        grid=(NPAD // BN,),
        in_specs=[
            pl.BlockSpec((BN, F_IN), lambda i: (i, 0)),
            pl.BlockSpec((F_IN, HC), lambda i: (0, 0)),
            pl.BlockSpec((HC, 16), lambda i: (0, 0)),
            pl.BlockSpec((HC, 16), lambda i: (0, 0)),
        ],
        out_specs=[
            pl.BlockSpec((BN, HC), lambda i: (i, 0)),
            pl.BlockSpec((BN, 16), lambda i: (i, 0)),
            pl.BlockSpec((BN, 16), lambda i: (i, 0)),
            pl.BlockSpec((2, 16), lambda i: (0, 0)),
        ],
        out_shape=[
            jax.ShapeDtypeStruct((NPAD, HC), jnp.float32),
            jax.ShapeDtypeStruct((NPAD, 16), jnp.float32),
            jax.ShapeDtypeStruct((NPAD, 16), jnp.float32),
            jax.ShapeDtypeStruct((2, 16), jnp.float32),
        ],
    )


def _make_prep2():
    nb = NPAD // BN
    return pl.pallas_call(
        _prep2_body,
        grid=(nb,),
        in_specs=[
            pl.BlockSpec((BN, HC), lambda i: (i, 0)),
            pl.BlockSpec((BN, HC), lambda i: (i + nb, 0)),
            pl.BlockSpec((1, HC), lambda i: (0, 0)),
            pl.BlockSpec((HC, NCLS), lambda i: (0, 0)),
            pl.BlockSpec((NCLS, 16), lambda i: (0, 0)),
            pl.BlockSpec((NCLS, 16), lambda i: (0, 0)),
        ],
        out_specs=[
            pl.BlockSpec((BN, NCLS), lambda i: (i, 0)),
            pl.BlockSpec((BN, 16), lambda i: (i, 0)),
            pl.BlockSpec((BN, 16), lambda i: (i, 0)),
            pl.BlockSpec((2, 16), lambda i: (0, 0)),
        ],
        out_shape=[
            jax.ShapeDtypeStruct((NPAD, NCLS), jnp.float32),
            jax.ShapeDtypeStruct((NPAD, 16), jnp.float32),
            jax.ShapeDtypeStruct((NPAD, 16), jnp.float32),
            jax.ShapeDtypeStruct((2, 16), jnp.float32),
        ],
    )


def _make_final():
    nb = NPAD // BN
    return pl.pallas_call(
        _final_body,
        grid=(nb,),
        in_specs=[
            pl.BlockSpec((BN, NCLS), lambda i: (i, 0)),
            pl.BlockSpec((BN, NCLS), lambda i: (i + nb, 0)),
            pl.BlockSpec((1, NCLS), lambda i: (0, 0)),
        ],
        out_specs=pl.BlockSpec((BN, NCLS), lambda i: (i, 0)),
        out_shape=jax.ShapeDtypeStruct((NPAD, NCLS), jnp.float32),
    )


def _make_passA(ept):
    nch = ept // CH

    @functools.partial(
        pl.kernel,
        out_type=jax.ShapeDtypeStruct((2 * NPAD, 16), jnp.float32),
        mesh=_mesh,
        compiler_params=pltpu.CompilerParams(use_tc_tiling_on_sc=False),
        scratch_types=[
            pltpu.VMEM((CH,), jnp.int32),
            pltpu.VMEM((CH,), jnp.int32),
            pltpu.VMEM((CH, 16), jnp.float32),
            pltpu.VMEM((CH, 16), jnp.float32),
            pltpu.VMEM((CH, 16), jnp.float32),
            pltpu.VMEM((2, 16), jnp.float32),
            pltpu.VMEM_SHARED((NPAD, 16), jnp.float32),
            pltpu.SemaphoreType.DMA,
            pltpu.SemaphoreType.DMA,
        ],
    )
    def passA(src_hbm, dst_hbm, as_hbm, ad_hbm, mk_hbm, dens_hbm,
              sbuf, dbuf, S, D, V, mkbuf, den_sh, sem1, sem2):
        c = lax.axis_index("c")
        s = lax.axis_index("s")
        tile_base = (c * NS + s) * ept
        row0 = s * RPT

        pltpu.sync_copy(mk_hbm, mkbuf)
        k = _k_from_mk(mkbuf)

        def zrow(i, _):
            V[i] = jnp.zeros((16,), jnp.float32)
            return _

        lax.fori_loop(0, CH, zrow, None)
        for j in range(RPT // CH):
            pltpu.sync_copy(V, den_sh.at[pl.ds(row0 + j * CH, CH)])
        plsc.subcore_barrier()

        def chunk_body(t, _):
            base = tile_base + t * CH
            pltpu.sync_copy(src_hbm.at[pl.ds(base, CH)], sbuf)
            pltpu.sync_copy(dst_hbm.at[pl.ds(base, CH)], dbuf)
            cp1 = pltpu.async_copy(as_hbm.at[sbuf], S, sem1)
            cp2 = pltpu.async_copy(ad_hbm.at[dbuf], D, sem2)
            cp1.wait()
            cp2.wait()

            def edge_body(i, _):
                e = S[i] + D[i]
                e = jnp.where(e >= 0.0, e, 0.2 * e)
                V[i] = jnp.exp(e - k)
                return _

            lax.fori_loop(0, CH, edge_body, None)
            pltpu.sync_copy(V, den_sh.at[dbuf], add=True)
            return _

        lax.fori_loop(0, nch, chunk_body, None)
        plsc.subcore_barrier()
        pltpu.sync_copy(den_sh.at[pl.ds(row0, RPT)],
                        dens_hbm.at[pl.ds(c * NPAD + row0, RPT)])

    return passA


def _make_passB(ept, heads, width):
    nch = ept // CH

    @functools.partial(
        pl.kernel,
        out_type=jax.ShapeDtypeStruct((2 * NPAD, width), jnp.float32),
        mesh=_mesh,
        compiler_params=pltpu.CompilerParams(use_tc_tiling_on_sc=False),
        scratch_types=[
            pltpu.VMEM((CH,), jnp.int32),
            pltpu.VMEM((CH,), jnp.int32),
            pltpu.VMEM((CH,), jnp.int32),
            pltpu.VMEM((CH, 16), jnp.float32),
            pltpu.VMEM((CH, 16), jnp.float32),
            pltpu.VMEM((CH, 16), jnp.float32),
            pltpu.VMEM((CH, 16), jnp.float32),
            pltpu.VMEM((CH, width), jnp.float32),
            pltpu.VMEM((2, 16), jnp.float32),
            pltpu.VMEM_SHARED((NPAD, width), jnp.float32),
            pltpu.SemaphoreType.DMA,
            pltpu.SemaphoreType.DMA,
            pltpu.SemaphoreType.DMA,
            pltpu.SemaphoreType.DMA,
            pltpu.SemaphoreType.DMA,
        ],
    )
    def passB(src_hbm, dst_hbm, as_hbm, ad_hbm, dens_hbm, ht_hbm, mk_hbm,
              outs_hbm,
              sbuf, dbuf, dbuf2, S, D, D0, D1, Hb, mkbuf, out_sh,
              sem1, sem2, sem3, sem4, sem5):
        c = lax.axis_index("c")
        s = lax.axis_index("s")
        tile_base = (c * NS + s) * ept
        row0 = s * RPT

        pltpu.sync_copy(mk_hbm, mkbuf)
        k = _k_from_mk(mkbuf)

        def zrow(i, _):
            for q in range(width // 16):
                Hb[i, pl.ds(q * 16, 16)] = jnp.zeros((16,), jnp.float32)
            return _

        lax.fori_loop(0, CH, zrow, None)
        for j in range(RPT // CH):
            pltpu.sync_copy(Hb, out_sh.at[pl.ds(row0 + j * CH, CH)])
        plsc.subcore_barrier()

        def chunk_body(t, _):
            base = tile_base + t * CH
            pltpu.sync_copy(src_hbm.at[pl.ds(base, CH)], sbuf)
            pltpu.sync_copy(dst_hbm.at[pl.ds(base, CH)], dbuf)

            def shift(j, _):
                dbuf2[pl.ds(j * 16, 16)] = dbuf[pl.ds(j * 16, 16)] + NPAD
                return _

            lax.fori_loop(0, CH // 16, shift, None)
            cp1 = pltpu.async_copy(as_hbm.at[sbuf], S, sem1)
            cp2 = pltpu.async_copy(ad_hbm.at[dbuf], D, sem2)
            cp3 = pltpu.async_copy(dens_hbm.at[dbuf], D0, sem3)
            cp4 = pltpu.async_copy(dens_hbm.at[dbuf2], D1, sem4)
            cp5 = pltpu.async_copy(ht_hbm.at[sbuf], Hb, sem5)
            cp1.wait()
            cp2.wait()
            cp3.wait()
            cp4.wait()
            cp5.wait()

            def edge_body(i, _):
                e = S[i] + D[i]
                e = jnp.where(e >= 0.0, e, 0.2 * e)
                val = jnp.exp(e - k)
                den = D0[i] + D1[i] + 1e-16
                al = val / den
                for h in range(heads):
                    sp = _splat(al, h)
                    for q in range((width // heads) // 16):
                        col = h * (width // heads) + q * 16
                        Hb[i, pl.ds(col, 16)] = Hb[i, pl.ds(col, 16)] * sp
                return _

            lax.fori_loop(0, CH, edge_body, None)
            pltpu.sync_copy(Hb, out_sh.at[dbuf], add=True)
            return _

        lax.fori_loop(0, nch, chunk_body, None)
        plsc.subcore_barrier()
        pltpu.sync_copy(out_sh.at[pl.ds(row0, RPT)],
                        outs_hbm.at[pl.ds(c * NPAD + row0, RPT)])

    return passB


def kernel(inputs, edge_index, edge_attr, W1, a_src1, a_dst1, b1,
           W2, a_src2, a_dst2, b2):
    del edge_attr
    E = edge_index.shape[1]
    etot = E + N
    ept = ((etot + NW * CH - 1) // (NW * CH)) * CH
    epad = ept * NW

    loop = jnp.arange(N, dtype=jnp.int32)
    npad_e = epad - etot
    trash = jnp.int32(N) + (jnp.arange(npad_e, dtype=jnp.int32) % 16)
    src_p = jnp.concatenate([edge_index[0], loop, trash])
    dst_p = jnp.concatenate([edge_index[1], loop, trash])

    x = jnp.pad(inputs, ((0, NPAD - N), (0, 0)))

    eye1 = jnp.eye(H1, dtype=jnp.float32)
    aws1 = jnp.pad((a_src1[:, :, None] * eye1[:, None, :]).reshape(HC, H1),
                   ((0, 0), (0, 16 - H1)))
    awd1 = jnp.pad((a_dst1[:, :, None] * eye1[:, None, :]).reshape(HC, H1),
                   ((0, 0), (0, 16 - H1)))
    aws2 = jnp.pad(a_src2.T, ((0, 0), (0, 15)))
    awd2 = jnp.pad(a_dst2.T, ((0, 0), (0, 15)))

    ht1, as1, ad1, mk1 = _make_prep1()(x, W1, aws1, awd1)
    dens1 = _make_passA(ept)(src_p, dst_p, as1, ad1, mk1)
    outs1 = _make_passB(ept, H1, HC)(src_p, dst_p, as1, ad1, dens1, ht1, mk1)
    ht2, as2, ad2, mk2 = _make_prep2()(outs1, outs1, b1.reshape(1, HC), W2,
                                       aws2, awd2)
    dens2 = _make_passA(ept)(src_p, dst_p, as2, ad2, mk2)
    outs2 = _make_passB(ept, 1, NCLS)(src_p, dst_p, as2, ad2, dens2, ht2, mk2)
    out = _make_final()(outs2, outs2, b2.reshape(1, NCLS))
    return out[:N]


# speedup vs baseline: 73.5610x; 1.8902x over previous
import functools

import jax
import jax.numpy as jnp
from jax import lax
from jax.experimental import pallas as pl
from jax.experimental.pallas import tpu as pltpu
from jax.experimental.pallas import tpu_sc as plsc

N = 10000
F_IN = 128
H1 = 8
HC = 128
NCLS = 32

NPAD = 10240
NC = 2
NS = 16
NW = NC * NS
CH = 64
RPT = NPAD // NS
BN = 1024

_mesh = plsc.VectorSubcoreMesh(
    core_axis_name="c", subcore_axis_name="s", num_cores=NC, num_subcores=NS)


def _splat(v, h):
    idx = jnp.full((16, 1), h, jnp.int32)
    dnums = lax.GatherDimensionNumbers(
        offset_dims=(), collapsed_slice_dims=(0,), start_index_map=(0,))
    return lax.gather(v, idx, dnums, (1,),
                      mode=lax.GatherScatterMode.PROMISE_IN_BOUNDS)


def _prep1_body(x_ref, w_ref, aws_ref, awd_ref, ht_ref, as_ref, ad_ref, mk_ref):
    i = pl.program_id(0)
    h = jnp.dot(x_ref[...], w_ref[...], preferred_element_type=jnp.float32)
    ht_ref[...] = h
    a_s = jnp.dot(h, aws_ref[...], preferred_element_type=jnp.float32)
    a_d = jnp.dot(h, awd_ref[...], preferred_element_type=jnp.float32)
    as_ref[...] = a_s
    ad_ref[...] = a_d
    bm = jnp.concatenate([jnp.full((1, 16), jnp.max(a_s), jnp.float32),
                          jnp.full((1, 16), jnp.max(a_d), jnp.float32)],
                         axis=0)

    @pl.when(i == 0)
    def _():
        mk_ref[...] = bm

    @pl.when(i > 0)
    def _():
        mk_ref[...] = jnp.maximum(mk_ref[...], bm)


def _prep2_body(o0_ref, o1_ref, d0_ref, d1_ref, e_ref, b_ref, w_ref,
                aws_ref, awd_ref, ht_ref, as_ref, ad_ref, mk_ref):
    i = pl.program_id(0)
    r = 1.0 / (d0_ref[...] + d1_ref[...] + 1e-16)
    rexp = jnp.dot(r, e_ref[...], preferred_element_type=jnp.float32)
    x2 = jax.nn.relu((o0_ref[...] + o1_ref[...]) * rexp + b_ref[...])
    h = jnp.dot(x2, w_ref[...], preferred_element_type=jnp.float32)
    ht_ref[...] = h
    a_s = jnp.dot(h, aws_ref[...], preferred_element_type=jnp.float32)
    a_d = jnp.dot(h, awd_ref[...], preferred_element_type=jnp.float32)
    as_ref[...] = a_s
    ad_ref[...] = a_d
    bm = jnp.concatenate([jnp.full((1, 16), jnp.max(a_s), jnp.float32),
                          jnp.full((1, 16), jnp.max(a_d), jnp.float32)],
                         axis=0)

    @pl.when(i == 0)
    def _():
        mk_ref[...] = bm

    @pl.when(i > 0)
    def _():
        mk_ref[...] = jnp.maximum(mk_ref[...], bm)


def _final_body(o0_ref, o1_ref, d0_ref, d1_ref, e_ref, b_ref, out_ref):
    r = 1.0 / (d0_ref[...] + d1_ref[...] + 1e-16)
    rexp = jnp.dot(r, e_ref[...], preferred_element_type=jnp.float32)
    out_ref[...] = (o0_ref[...] + o1_ref[...]) * rexp + b_ref[...]


def _make_prep1():
    return pl.pallas_call(
        _prep1_body,
        grid=(NPAD // BN,),
        in_specs=[
            pl.BlockSpec((BN, F_IN), lambda i: (i, 0)),
            pl.BlockSpec((F_IN, HC), lambda i: (0, 0)),
            pl.BlockSpec((HC, 16), lambda i: (0, 0)),
            pl.BlockSpec((HC, 16), lambda i: (0, 0)),
        ],
        out_specs=[
            pl.BlockSpec((BN, HC), lambda i: (i, 0)),
            pl.BlockSpec((BN, 16), lambda i: (i, 0)),
            pl.BlockSpec((BN, 16), lambda i: (i, 0)),
            pl.BlockSpec((2, 16), lambda i: (0, 0)),
        ],
        out_shape=[
            jax.ShapeDtypeStruct((NPAD, HC), jnp.float32),
            jax.ShapeDtypeStruct((NPAD, 16), jnp.float32),
            jax.ShapeDtypeStruct((NPAD, 16), jnp.float32),
            jax.ShapeDtypeStruct((2, 16), jnp.float32),
        ],
    )


def _make_prep2():
    nb = NPAD // BN
    return pl.pallas_call(
        _prep2_body,
        grid=(nb,),
        in_specs=[
            pl.BlockSpec((BN, HC), lambda i: (i, 0)),
            pl.BlockSpec((BN, HC), lambda i: (i + nb, 0)),
            pl.BlockSpec((BN, 16), lambda i: (i, 0)),
            pl.BlockSpec((BN, 16), lambda i: (i + nb, 0)),
            pl.BlockSpec((16, HC), lambda i: (0, 0)),
            pl.BlockSpec((1, HC), lambda i: (0, 0)),
            pl.BlockSpec((HC, NCLS), lambda i: (0, 0)),
            pl.BlockSpec((NCLS, 16), lambda i: (0, 0)),
            pl.BlockSpec((NCLS, 16), lambda i: (0, 0)),
        ],
        out_specs=[
            pl.BlockSpec((BN, NCLS), lambda i: (i, 0)),
            pl.BlockSpec((BN, 16), lambda i: (i, 0)),
            pl.BlockSpec((BN, 16), lambda i: (i, 0)),
            pl.BlockSpec((2, 16), lambda i: (0, 0)),
        ],
        out_shape=[
            jax.ShapeDtypeStruct((NPAD, NCLS), jnp.float32),
            jax.ShapeDtypeStruct((NPAD, 16), jnp.float32),
            jax.ShapeDtypeStruct((NPAD, 16), jnp.float32),
            jax.ShapeDtypeStruct((2, 16), jnp.float32),
        ],
    )


def _make_final():
    nb = NPAD // BN
    return pl.pallas_call(
        _final_body,
        grid=(nb,),
        in_specs=[
            pl.BlockSpec((BN, NCLS), lambda i: (i, 0)),
            pl.BlockSpec((BN, NCLS), lambda i: (i + nb, 0)),
            pl.BlockSpec((BN, 16), lambda i: (i, 0)),
            pl.BlockSpec((BN, 16), lambda i: (i + nb, 0)),
            pl.BlockSpec((16, NCLS), lambda i: (0, 0)),
            pl.BlockSpec((1, NCLS), lambda i: (0, 0)),
        ],
        out_specs=pl.BlockSpec((BN, NCLS), lambda i: (i, 0)),
        out_shape=jax.ShapeDtypeStruct((NPAD, NCLS), jnp.float32),
    )


def _make_edge(ept, heads, width):
    nch = ept // CH

    @functools.partial(
        pl.kernel,
        out_type=(jax.ShapeDtypeStruct((2 * NPAD, 16), jnp.float32),
                  jax.ShapeDtypeStruct((2 * NPAD, width), jnp.float32)),
        mesh=_mesh,
        compiler_params=pltpu.CompilerParams(use_tc_tiling_on_sc=False),
        scratch_types=[
            pltpu.VMEM((nch, CH), jnp.int32),
            pltpu.VMEM((CH,), jnp.int32),
            pltpu.VMEM((CH,), jnp.int32),
            pltpu.VMEM((CH,), jnp.int32),
            pltpu.VMEM((CH,), jnp.int32),
            pltpu.VMEM((CH, 16), jnp.float32),
            pltpu.VMEM((CH, 16), jnp.float32),
            pltpu.VMEM((CH, 16), jnp.float32),
            pltpu.VMEM((CH, width), jnp.float32),
            pltpu.VMEM((CH, 16), jnp.float32),
            pltpu.VMEM((CH, 16), jnp.float32),
            pltpu.VMEM((CH, 16), jnp.float32),
            pltpu.VMEM((CH, width), jnp.float32),
            pltpu.VMEM((2, 16), jnp.float32),
            pltpu.VMEM_SHARED((NPAD, 16), jnp.float32),
            pltpu.VMEM_SHARED((NPAD, width), jnp.float32),
            pltpu.SemaphoreType.DMA,
            pltpu.SemaphoreType.DMA,
        ],
    )
    def edge(pk_hbm, as_hbm, ad_hbm, ht_hbm, mk_hbm, dens_hbm, outs_hbm,
             pbig, sbuf_a, dbuf_a, sbuf_b, dbuf_b,
             S_a, D_a, V_a, Hb_a, S_b, D_b, V_b, Hb_b,
             mkbuf, den_sh, out_sh, sem_a, sem_b):
        c = lax.axis_index("c")
        s = lax.axis_index("s")
        chunk0 = (c * NS + s) * nch
        row0 = s * RPT

        pltpu.sync_copy(mk_hbm, mkbuf)
        k = mkbuf[0] + mkbuf[1]
        pltpu.sync_copy(pk_hbm.at[pl.ds(chunk0, nch)], pbig)

        def zrow(i, _):
            V_a[i] = jnp.zeros((16,), jnp.float32)
            for q in range(width // 16):
                Hb_a[i, pl.ds(q * 16, 16)] = jnp.zeros((16,), jnp.float32)
            return _

        lax.fori_loop(0, CH, zrow, None)
        for j in range(RPT // CH):
            pltpu.sync_copy(V_a, den_sh.at[pl.ds(row0 + j * CH, CH)])
            pltpu.sync_copy(Hb_a, out_sh.at[pl.ds(row0 + j * CH, CH)])
        plsc.subcore_barrier()

        def issue(t, sbuf, dbuf, S, D, Hb, sem):
            def ub(j, _):
                p = pbig[t, pl.ds(j * 16, 16)]
                sbuf[pl.ds(j * 16, 16)] = p & 0xFFFF
                dbuf[pl.ds(j * 16, 16)] = p >> 16
                return _

            lax.fori_loop(0, CH // 16, ub, None, unroll=CH // 16)
            pltpu.async_copy(as_hbm.at[sbuf], S, sem)
            pltpu.async_copy(ad_hbm.at[dbuf], D, sem)
            pltpu.async_copy(ht_hbm.at[sbuf], Hb, sem)

        def compute(sbuf, dbuf, S, D, V, Hb, sem):
            pltpu.make_async_copy(as_hbm.at[sbuf], S, sem).wait()
            pltpu.make_async_copy(ad_hbm.at[dbuf], D, sem).wait()
            pltpu.make_async_copy(ht_hbm.at[sbuf], Hb, sem).wait()

            def edge_body(i, _):
                e = S[i] + D[i]
                e = jnp.where(e >= 0.0, e, 0.2 * e)
                val = jnp.exp(e - k)
                V[i] = val
                for h in range(heads):
                    sp = _splat(val, h)
                    for q in range((width // heads) // 16):
                        col = h * (width // heads) + q * 16
                        Hb[i, pl.ds(col, 16)] = Hb[i, pl.ds(col, 16)] * sp
                return _

            lax.fori_loop(0, CH, edge_body, None, unroll=2)
            pltpu.sync_copy(V, den_sh.at[dbuf], add=True)
            pltpu.sync_copy(Hb, out_sh.at[dbuf], add=True)

        issue(0, sbuf_a, dbuf_a, S_a, D_a, Hb_a, sem_a)

        def pair_body(tt, _):
            t0 = 2 * tt
            issue(t0 + 1, sbuf_b, dbuf_b, S_b, D_b, Hb_b, sem_b)
            compute(sbuf_a, dbuf_a, S_a, D_a, V_a, Hb_a, sem_a)

            @pl.when(tt < nch // 2 - 1)
            def _():
                issue(t0 + 2, sbuf_a, dbuf_a, S_a, D_a, Hb_a, sem_a)

            compute(sbuf_b, dbuf_b, S_b, D_b, V_b, Hb_b, sem_b)
            return _

        lax.fori_loop(0, nch // 2, pair_body, None)
        plsc.subcore_barrier()
        pltpu.sync_copy(den_sh.at[pl.ds(row0, RPT)],
                        dens_hbm.at[pl.ds(c * NPAD + row0, RPT)])
        pltpu.sync_copy(out_sh.at[pl.ds(row0, RPT)],
                        outs_hbm.at[pl.ds(c * NPAD + row0, RPT)])

    return edge


def kernel(inputs, edge_index, edge_attr, W1, a_src1, a_dst1, b1,
           W2, a_src2, a_dst2, b2):
    del edge_attr
    E = edge_index.shape[1]
    etot = E + N
    ept = ((etot + NW * 2 * CH - 1) // (NW * 2 * CH)) * 2 * CH
    epad = ept * NW

    loop = jnp.arange(N, dtype=jnp.int32)
    npad_e = epad - etot
    trash = jnp.int32(N) + (jnp.arange(npad_e, dtype=jnp.int32) % 16)
    src_p = jnp.concatenate([edge_index[0], loop, trash])
    dst_p = jnp.concatenate([edge_index[1], loop, trash])
    packed = (dst_p * jnp.int32(65536) + src_p).reshape(-1, CH)

    x = jnp.pad(inputs, ((0, NPAD - N), (0, 0)))

    eye1 = jnp.eye(H1, dtype=jnp.float32)
    aws1 = jnp.pad((a_src1[:, :, None] * eye1[:, None, :]).reshape(HC, H1),
                   ((0, 0), (0, 16 - H1)))
    awd1 = jnp.pad((a_dst1[:, :, None] * eye1[:, None, :]).reshape(HC, H1),
                   ((0, 0), (0, 16 - H1)))
    aws2 = jnp.pad(a_src2.T, ((0, 0), (0, 15)))
    awd2 = jnp.pad(a_dst2.T, ((0, 0), (0, 15)))

    lanes = jnp.arange(16)
    exp1 = (lanes[:, None] == (jnp.arange(HC) // (HC // H1))[None, :])
    exp1 = exp1.astype(jnp.float32)
    exp2 = (lanes[:, None] == jnp.zeros((NCLS,), jnp.int32)[None, :])
    exp2 = exp2.astype(jnp.float32)

    ht1, as1, ad1, mk1 = _make_prep1()(x, W1, aws1, awd1)
    dens1, outs1 = _make_edge(ept, H1, HC)(packed, as1, ad1, ht1, mk1)
    ht2, as2, ad2, mk2 = _make_prep2()(outs1, outs1, dens1, dens1, exp1,
                                       b1.reshape(1, HC), W2, aws2, awd2)
    dens2, outs2 = _make_edge(ept, 1, NCLS)(packed, as2, ad2, ht2, mk2)
    out = _make_final()(outs2, outs2, dens2, dens2, exp2,
                        b2.reshape(1, NCLS))
    return out[:N]


# speedup vs baseline: 75.7487x; 1.0297x over previous
import functools

import jax
import jax.numpy as jnp
from jax import lax
from jax.experimental import pallas as pl
from jax.experimental.pallas import tpu as pltpu
from jax.experimental.pallas import tpu_sc as plsc

N = 10000
F_IN = 128
H1 = 8
HC = 128
NCLS = 32

NPAD = 10240
NC = 2
NS = 16
NW = NC * NS
CH = 64
RPT = NPAD // NS
BN = 1024

_mesh = plsc.VectorSubcoreMesh(
    core_axis_name="c", subcore_axis_name="s", num_cores=NC, num_subcores=NS)


def _splat(v, h):
    idx = jnp.full((16, 1), h, jnp.int32)
    dnums = lax.GatherDimensionNumbers(
        offset_dims=(), collapsed_slice_dims=(0,), start_index_map=(0,))
    return lax.gather(v, idx, dnums, (1,),
                      mode=lax.GatherScatterMode.PROMISE_IN_BOUNDS)


def _prep1_body(x_ref, w_ref, aws_ref, awd_ref, ht_ref, as_ref, ad_ref, mk_ref):
    i = pl.program_id(0)
    h = jnp.dot(x_ref[...], w_ref[...], preferred_element_type=jnp.float32)
    ht_ref[...] = h
    a_s = jnp.dot(h, aws_ref[...], preferred_element_type=jnp.float32)
    a_d = jnp.dot(h, awd_ref[...], preferred_element_type=jnp.float32)
    as_ref[...] = a_s
    ad_ref[...] = a_d
    bm = jnp.concatenate([jnp.full((1, 16), jnp.max(a_s), jnp.float32),
                          jnp.full((1, 16), jnp.max(a_d), jnp.float32)],
                         axis=0)

    @pl.when(i == 0)
    def _():
        mk_ref[...] = bm

    @pl.when(i > 0)
    def _():
        mk_ref[...] = jnp.maximum(mk_ref[...], bm)


def _prep2_body(o0_ref, o1_ref, d0_ref, d1_ref, e_ref, b_ref, w_ref,
                aws_ref, awd_ref, ht_ref, as_ref, ad_ref, mk_ref):
    i = pl.program_id(0)
    r = 1.0 / (d0_ref[...] + d1_ref[...] + 1e-16)
    rexp = jnp.dot(r, e_ref[...], preferred_element_type=jnp.float32)
    x2 = jax.nn.relu((o0_ref[...] + o1_ref[...]) * rexp + b_ref[...])
    h = jnp.dot(x2, w_ref[...], preferred_element_type=jnp.float32)
    ht_ref[...] = h
    a_s = jnp.dot(h, aws_ref[...], preferred_element_type=jnp.float32)
    a_d = jnp.dot(h, awd_ref[...], preferred_element_type=jnp.float32)
    as_ref[...] = a_s
    ad_ref[...] = a_d
    bm = jnp.concatenate([jnp.full((1, 16), jnp.max(a_s), jnp.float32),
                          jnp.full((1, 16), jnp.max(a_d), jnp.float32)],
                         axis=0)

    @pl.when(i == 0)
    def _():
        mk_ref[...] = bm

    @pl.when(i > 0)
    def _():
        mk_ref[...] = jnp.maximum(mk_ref[...], bm)


def _final_body(o0_ref, o1_ref, d0_ref, d1_ref, e_ref, b_ref, out_ref):
    r = 1.0 / (d0_ref[...] + d1_ref[...] + 1e-16)
    rexp = jnp.dot(r, e_ref[...], preferred_element_type=jnp.float32)
    out_ref[...] = (o0_ref[...] + o1_ref[...]) * rexp + b_ref[...]


def _make_prep1():
    return pl.pallas_call(
        _prep1_body,
        grid=(NPAD // BN,),
        in_specs=[
            pl.BlockSpec((BN, F_IN), lambda i: (i, 0)),
            pl.BlockSpec((F_IN, HC), lambda i: (0, 0)),
            pl.BlockSpec((HC, 16), lambda i: (0, 0)),
            pl.BlockSpec((HC, 16), lambda i: (0, 0)),
        ],
        out_specs=[
            pl.BlockSpec((BN, HC), lambda i: (i, 0)),
            pl.BlockSpec((BN, 16), lambda i: (i, 0)),
            pl.BlockSpec((BN, 16), lambda i: (i, 0)),
            pl.BlockSpec((2, 16), lambda i: (0, 0)),
        ],
        out_shape=[
            jax.ShapeDtypeStruct((NPAD, HC), jnp.float32),
            jax.ShapeDtypeStruct((NPAD, 16), jnp.float32),
            jax.ShapeDtypeStruct((NPAD, 16), jnp.float32),
            jax.ShapeDtypeStruct((2, 16), jnp.float32),
        ],
    )


def _make_prep2():
    nb = NPAD // BN
    return pl.pallas_call(
        _prep2_body,
        grid=(nb,),
        in_specs=[
            pl.BlockSpec((BN, HC), lambda i: (i, 0)),
            pl.BlockSpec((BN, HC), lambda i: (i + nb, 0)),
            pl.BlockSpec((BN, 16), lambda i: (i, 0)),
            pl.BlockSpec((BN, 16), lambda i: (i + nb, 0)),
            pl.BlockSpec((16, HC), lambda i: (0, 0)),
            pl.BlockSpec((1, HC), lambda i: (0, 0)),
            pl.BlockSpec((HC, NCLS), lambda i: (0, 0)),
            pl.BlockSpec((NCLS, 16), lambda i: (0, 0)),
            pl.BlockSpec((NCLS, 16), lambda i: (0, 0)),
        ],
        out_specs=[
            pl.BlockSpec((BN, NCLS), lambda i: (i, 0)),
            pl.BlockSpec((BN, 16), lambda i: (i, 0)),
            pl.BlockSpec((BN, 16), lambda i: (i, 0)),
            pl.BlockSpec((2, 16), lambda i: (0, 0)),
        ],
        out_shape=[
            jax.ShapeDtypeStruct((NPAD, NCLS), jnp.float32),
            jax.ShapeDtypeStruct((NPAD, 16), jnp.float32),
            jax.ShapeDtypeStruct((NPAD, 16), jnp.float32),
            jax.ShapeDtypeStruct((2, 16), jnp.float32),
        ],
    )


def _make_final():
    nb = NPAD // BN
    return pl.pallas_call(
        _final_body,
        grid=(nb,),
        in_specs=[
            pl.BlockSpec((BN, NCLS), lambda i: (i, 0)),
            pl.BlockSpec((BN, NCLS), lambda i: (i + nb, 0)),
            pl.BlockSpec((BN, 16), lambda i: (i, 0)),
            pl.BlockSpec((BN, 16), lambda i: (i + nb, 0)),
            pl.BlockSpec((16, NCLS), lambda i: (0, 0)),
            pl.BlockSpec((1, NCLS), lambda i: (0, 0)),
        ],
        out_specs=pl.BlockSpec((BN, NCLS), lambda i: (i, 0)),
        out_shape=jax.ShapeDtypeStruct((NPAD, NCLS), jnp.float32),
    )


def _make_edge(ept, heads, width):
    nch = ept // CH

    @functools.partial(
        pl.kernel,
        out_type=(jax.ShapeDtypeStruct((2 * NPAD, 16), jnp.float32),
                  jax.ShapeDtypeStruct((2 * NPAD, width), jnp.float32)),
        mesh=_mesh,
        compiler_params=pltpu.CompilerParams(use_tc_tiling_on_sc=False),
        scratch_types=[
            pltpu.VMEM((nch, CH), jnp.int32),
            pltpu.VMEM((CH,), jnp.int32),
            pltpu.VMEM((CH,), jnp.int32),
            pltpu.VMEM((CH,), jnp.int32),
            pltpu.VMEM((CH,), jnp.int32),
            pltpu.VMEM((CH, 16), jnp.float32),
            pltpu.VMEM((CH, 16), jnp.float32),
            pltpu.VMEM((CH, 16), jnp.float32),
            pltpu.VMEM((CH, width), jnp.float32),
            pltpu.VMEM((CH, 16), jnp.float32),
            pltpu.VMEM((CH, 16), jnp.float32),
            pltpu.VMEM((CH, 16), jnp.float32),
            pltpu.VMEM((CH, width), jnp.float32),
            pltpu.VMEM((2, 16), jnp.float32),
            pltpu.VMEM_SHARED((NPAD, 16), jnp.float32),
            pltpu.VMEM_SHARED((NPAD, width), jnp.float32),
            pltpu.SemaphoreType.DMA,
            pltpu.SemaphoreType.DMA,
        ],
    )
    def edge(pk_hbm, as_hbm, ad_hbm, ht_hbm, mk_hbm, dens_hbm, outs_hbm,
             pbig, sbuf_a, dbuf_a, sbuf_b, dbuf_b,
             S_a, D_a, V_a, Hb_a, S_b, D_b, V_b, Hb_b,
             mkbuf, den_sh, out_sh, sem_a, sem_b):
        c = lax.axis_index("c")
        s = lax.axis_index("s")
        chunk0 = (c * NS + s) * nch
        row0 = s * RPT

        pltpu.sync_copy(mk_hbm, mkbuf)
        k = mkbuf[0] + mkbuf[1]
        pltpu.sync_copy(pk_hbm.at[pl.ds(chunk0, nch)], pbig)

        def zrow(i, _):
            V_a[i] = jnp.zeros((16,), jnp.float32)
            for q in range(width // 16):
                Hb_a[i, pl.ds(q * 16, 16)] = jnp.zeros((16,), jnp.float32)
            return _

        lax.fori_loop(0, CH, zrow, None)
        for j in range(RPT // CH):
            pltpu.sync_copy(V_a, den_sh.at[pl.ds(row0 + j * CH, CH)])
            pltpu.sync_copy(Hb_a, out_sh.at[pl.ds(row0 + j * CH, CH)])
        plsc.subcore_barrier()

        def issue(t, sbuf, dbuf, S, D, Hb, sem):
            def ub(j, _):
                p = pbig[t, pl.ds(j * 16, 16)]
                sbuf[pl.ds(j * 16, 16)] = p & 0xFFFF
                dbuf[pl.ds(j * 16, 16)] = p >> 16
                return _

            lax.fori_loop(0, CH // 16, ub, None, unroll=CH // 16)
            pltpu.async_copy(as_hbm.at[sbuf], S, sem)
            pltpu.async_copy(ad_hbm.at[dbuf], D, sem)
            pltpu.async_copy(ht_hbm.at[sbuf], Hb, sem)

        def compute(sbuf, dbuf, S, D, V, Hb, sem):
            pltpu.make_async_copy(as_hbm.at[sbuf], S, sem).wait()
            pltpu.make_async_copy(ad_hbm.at[dbuf], D, sem).wait()
            pltpu.make_async_copy(ht_hbm.at[sbuf], Hb, sem).wait()

            def edge_body(i, _):
                e = S[i] + D[i]
                e = jnp.where(e >= 0.0, e, 0.2 * e)
                val = jnp.exp(e - k)
                V[i] = val
                for h in range(heads):
                    sp = _splat(val, h)
                    for q in range((width // heads) // 16):
                        col = h * (width // heads) + q * 16
                        Hb[i, pl.ds(col, 16)] = Hb[i, pl.ds(col, 16)] * sp
                return _

            lax.fori_loop(0, CH, edge_body, None, unroll=2)
            cpv = pltpu.async_copy(V, den_sh.at[dbuf], sem, add=True)
            cph = pltpu.async_copy(Hb, out_sh.at[dbuf], sem, add=True)
            cpv.wait()
            cph.wait()

        issue(0, sbuf_a, dbuf_a, S_a, D_a, Hb_a, sem_a)

        def pair_body(tt, _):
            t0 = 2 * tt
            issue(t0 + 1, sbuf_b, dbuf_b, S_b, D_b, Hb_b, sem_b)
            compute(sbuf_a, dbuf_a, S_a, D_a, V_a, Hb_a, sem_a)

            @pl.when(tt < nch // 2 - 1)
            def _():
                issue(t0 + 2, sbuf_a, dbuf_a, S_a, D_a, Hb_a, sem_a)

            compute(sbuf_b, dbuf_b, S_b, D_b, V_b, Hb_b, sem_b)
            return _

        lax.fori_loop(0, nch // 2, pair_body, None)
        plsc.subcore_barrier()
        pltpu.sync_copy(den_sh.at[pl.ds(row0, RPT)],
                        dens_hbm.at[pl.ds(c * NPAD + row0, RPT)])
        pltpu.sync_copy(out_sh.at[pl.ds(row0, RPT)],
                        outs_hbm.at[pl.ds(c * NPAD + row0, RPT)])

    return edge


def kernel(inputs, edge_index, edge_attr, W1, a_src1, a_dst1, b1,
           W2, a_src2, a_dst2, b2):
    del edge_attr
    E = edge_index.shape[1]
    etot = E + N
    ept = ((etot + NW * 2 * CH - 1) // (NW * 2 * CH)) * 2 * CH
    epad = ept * NW

    loop = jnp.arange(N, dtype=jnp.int32)
    npad_e = epad - etot
    trash = jnp.int32(N) + (jnp.arange(npad_e, dtype=jnp.int32) % 16)
    src_p = jnp.concatenate([edge_index[0], loop, trash])
    dst_p = jnp.concatenate([edge_index[1], loop, trash])
    packed = (dst_p * jnp.int32(65536) + src_p).reshape(-1, CH)

    x = jnp.pad(inputs, ((0, NPAD - N), (0, 0)))

    eye1 = jnp.eye(H1, dtype=jnp.float32)
    aws1 = jnp.pad((a_src1[:, :, None] * eye1[:, None, :]).reshape(HC, H1),
                   ((0, 0), (0, 16 - H1)))
    awd1 = jnp.pad((a_dst1[:, :, None] * eye1[:, None, :]).reshape(HC, H1),
                   ((0, 0), (0, 16 - H1)))
    aws2 = jnp.pad(a_src2.T, ((0, 0), (0, 15)))
    awd2 = jnp.pad(a_dst2.T, ((0, 0), (0, 15)))

    lanes = jnp.arange(16)
    exp1 = (lanes[:, None] == (jnp.arange(HC) // (HC // H1))[None, :])
    exp1 = exp1.astype(jnp.float32)
    exp2 = (lanes[:, None] == jnp.zeros((NCLS,), jnp.int32)[None, :])
    exp2 = exp2.astype(jnp.float32)

    ht1, as1, ad1, mk1 = _make_prep1()(x, W1, aws1, awd1)
    dens1, outs1 = _make_edge(ept, H1, HC)(packed, as1, ad1, ht1, mk1)
    ht2, as2, ad2, mk2 = _make_prep2()(outs1, outs1, dens1, dens1, exp1,
                                       b1.reshape(1, HC), W2, aws2, awd2)
    dens2, outs2 = _make_edge(ept, 1, NCLS)(packed, as2, ad2, ht2, mk2)
    out = _make_final()(outs2, outs2, dens2, dens2, exp2,
                        b2.reshape(1, NCLS))
    return out[:N]


# speedup vs baseline: 128.7201x; 1.6993x over previous
import functools

import jax
import jax.numpy as jnp
from jax import lax
from jax.experimental import pallas as pl
from jax.experimental.pallas import tpu as pltpu
from jax.experimental.pallas import tpu_sc as plsc

N = 10000
F_IN = 128
H1 = 8
HC = 128
NCLS = 32

NPAD = 10240
NC = 2
NS = 16
NW = NC * NS
CH = 64
RPT = NPAD // NS
BN = 1024

_mesh = plsc.VectorSubcoreMesh(
    core_axis_name="c", subcore_axis_name="s", num_cores=NC, num_subcores=NS)


def _splat(v, h):
    idx = jnp.full((16, 1), h, jnp.int32)
    dnums = lax.GatherDimensionNumbers(
        offset_dims=(), collapsed_slice_dims=(0,), start_index_map=(0,))
    return lax.gather(v, idx, dnums, (1,),
                      mode=lax.GatherScatterMode.PROMISE_IN_BOUNDS)


def _prep1_body(x_ref, w_ref, aws_ref, awd_ref, ht_ref, as_ref, ad_ref, mk_ref):
    i = pl.program_id(0)
    h = jnp.dot(x_ref[...], w_ref[...], preferred_element_type=jnp.float32)
    ht_ref[...] = h
    a_s = jnp.dot(h, aws_ref[...], preferred_element_type=jnp.float32)
    a_d = jnp.dot(h, awd_ref[...], preferred_element_type=jnp.float32)
    as_ref[...] = a_s
    ad_ref[...] = a_d
    bm = jnp.concatenate([jnp.full((1, 16), jnp.max(a_s), jnp.float32),
                          jnp.full((1, 16), jnp.max(a_d), jnp.float32)],
                         axis=0)

    @pl.when(i == 0)
    def _():
        mk_ref[...] = bm

    @pl.when(i > 0)
    def _():
        mk_ref[...] = jnp.maximum(mk_ref[...], bm)


def _prep2_body(o0_ref, o1_ref, d0_ref, d1_ref, e_ref, b_ref, w_ref,
                aws_ref, awd_ref, ht_ref, as_ref, ad_ref, mk_ref):
    i = pl.program_id(0)
    r = 1.0 / (d0_ref[...] + d1_ref[...] + 1e-16)
    rexp = jnp.dot(r, e_ref[...], preferred_element_type=jnp.float32)
    x2 = jax.nn.relu((o0_ref[...] + o1_ref[...]) * rexp + b_ref[...])
    h = jnp.dot(x2, w_ref[...], preferred_element_type=jnp.float32)
    ht_ref[...] = h
    a_s = jnp.dot(h, aws_ref[...], preferred_element_type=jnp.float32)
    a_d = jnp.dot(h, awd_ref[...], preferred_element_type=jnp.float32)
    as_ref[...] = a_s
    ad_ref[...] = a_d
    bm = jnp.concatenate([jnp.full((1, 16), jnp.max(a_s), jnp.float32),
                          jnp.full((1, 16), jnp.max(a_d), jnp.float32)],
                         axis=0)

    @pl.when(i == 0)
    def _():
        mk_ref[...] = bm

    @pl.when(i > 0)
    def _():
        mk_ref[...] = jnp.maximum(mk_ref[...], bm)


def _final_body(o0_ref, o1_ref, d0_ref, d1_ref, e_ref, b_ref, out_ref):
    r = 1.0 / (d0_ref[...] + d1_ref[...] + 1e-16)
    rexp = jnp.dot(r, e_ref[...], preferred_element_type=jnp.float32)
    out_ref[...] = (o0_ref[...] + o1_ref[...]) * rexp + b_ref[...]


def _make_prep1():
    return pl.pallas_call(
        _prep1_body,
        grid=(NPAD // BN,),
        in_specs=[
            pl.BlockSpec((BN, F_IN), lambda i: (i, 0)),
            pl.BlockSpec((F_IN, HC), lambda i: (0, 0)),
            pl.BlockSpec((HC, 16), lambda i: (0, 0)),
            pl.BlockSpec((HC, 16), lambda i: (0, 0)),
        ],
        out_specs=[
            pl.BlockSpec((BN, HC), lambda i: (i, 0)),
            pl.BlockSpec((BN, 16), lambda i: (i, 0)),
            pl.BlockSpec((BN, 16), lambda i: (i, 0)),
            pl.BlockSpec((2, 16), lambda i: (0, 0)),
        ],
        out_shape=[
            jax.ShapeDtypeStruct((NPAD, HC), jnp.float32),
            jax.ShapeDtypeStruct((NPAD, 16), jnp.float32),
            jax.ShapeDtypeStruct((NPAD, 16), jnp.float32),
            jax.ShapeDtypeStruct((2, 16), jnp.float32),
        ],
    )


def _make_prep2():
    nb = NPAD // BN
    return pl.pallas_call(
        _prep2_body,
        grid=(nb,),
        in_specs=[
            pl.BlockSpec((BN, HC), lambda i: (i, 0)),
            pl.BlockSpec((BN, HC), lambda i: (i + nb, 0)),
            pl.BlockSpec((BN, 16), lambda i: (i, 0)),
            pl.BlockSpec((BN, 16), lambda i: (i + nb, 0)),
            pl.BlockSpec((16, HC), lambda i: (0, 0)),
            pl.BlockSpec((1, HC), lambda i: (0, 0)),
            pl.BlockSpec((HC, NCLS), lambda i: (0, 0)),
            pl.BlockSpec((NCLS, 16), lambda i: (0, 0)),
            pl.BlockSpec((NCLS, 16), lambda i: (0, 0)),
        ],
        out_specs=[
            pl.BlockSpec((BN, NCLS), lambda i: (i, 0)),
            pl.BlockSpec((BN, 16), lambda i: (i, 0)),
            pl.BlockSpec((BN, 16), lambda i: (i, 0)),
            pl.BlockSpec((2, 16), lambda i: (0, 0)),
        ],
        out_shape=[
            jax.ShapeDtypeStruct((NPAD, NCLS), jnp.float32),
            jax.ShapeDtypeStruct((NPAD, 16), jnp.float32),
            jax.ShapeDtypeStruct((NPAD, 16), jnp.float32),
            jax.ShapeDtypeStruct((2, 16), jnp.float32),
        ],
    )


def _make_final():
    nb = NPAD // BN
    return pl.pallas_call(
        _final_body,
        grid=(nb,),
        in_specs=[
            pl.BlockSpec((BN, NCLS), lambda i: (i, 0)),
            pl.BlockSpec((BN, NCLS), lambda i: (i + nb, 0)),
            pl.BlockSpec((BN, 16), lambda i: (i, 0)),
            pl.BlockSpec((BN, 16), lambda i: (i + nb, 0)),
            pl.BlockSpec((16, NCLS), lambda i: (0, 0)),
            pl.BlockSpec((1, NCLS), lambda i: (0, 0)),
        ],
        out_specs=pl.BlockSpec((BN, NCLS), lambda i: (i, 0)),
        out_shape=jax.ShapeDtypeStruct((NPAD, NCLS), jnp.float32),
    )


def _make_edge(ept, heads, width):
    nch = ept // CH

    @functools.partial(
        pl.kernel,
        out_type=(jax.ShapeDtypeStruct((2 * NPAD, 16), jnp.float32),
                  jax.ShapeDtypeStruct((2 * NPAD, width), jnp.float32)),
        mesh=_mesh,
        compiler_params=pltpu.CompilerParams(use_tc_tiling_on_sc=False),
        scratch_types=[
            pltpu.VMEM((nch, CH), jnp.int32),
            pltpu.VMEM((CH,), jnp.int32),
            pltpu.VMEM((CH,), jnp.int32),
            pltpu.VMEM((CH,), jnp.int32),
            pltpu.VMEM((CH,), jnp.int32),
            pltpu.VMEM((CH, 16), jnp.float32),
            pltpu.VMEM((CH, 16), jnp.float32),
            pltpu.VMEM((CH, 16), jnp.float32),
            pltpu.VMEM((CH, width), jnp.float32),
            pltpu.VMEM((CH, 16), jnp.float32),
            pltpu.VMEM((CH, 16), jnp.float32),
            pltpu.VMEM((CH, 16), jnp.float32),
            pltpu.VMEM((CH, width), jnp.float32),
            pltpu.VMEM((2, 16), jnp.float32),
            pltpu.VMEM_SHARED((NPAD, 16), jnp.float32),
            pltpu.VMEM_SHARED((NPAD, width), jnp.float32),
            pltpu.SemaphoreType.DMA,
            pltpu.SemaphoreType.DMA,
        ],
    )
    def edge(pk_hbm, as_hbm, ad_hbm, ht_hbm, mk_hbm, dens_hbm, outs_hbm,
             pbig, sbuf_a, dbuf_a, sbuf_b, dbuf_b,
             S_a, D_a, V_a, Hb_a, S_b, D_b, V_b, Hb_b,
             mkbuf, den_sh, out_sh, sem_a, sem_b):
        c = lax.axis_index("c")
        s = lax.axis_index("s")
        chunk0 = (c * NS + s) * nch
        row0 = s * RPT

        pltpu.sync_copy(mk_hbm, mkbuf)
        k = mkbuf[0] + mkbuf[1]
        pltpu.sync_copy(pk_hbm.at[pl.ds(chunk0, nch)], pbig)

        def zrow(i, _):
            V_a[i] = jnp.zeros((16,), jnp.float32)
            for q in range(width // 16):
                Hb_a[i, pl.ds(q * 16, 16)] = jnp.zeros((16,), jnp.float32)
            return _

        lax.fori_loop(0, CH, zrow, None)
        for j in range(RPT // CH):
            pltpu.sync_copy(V_a, den_sh.at[pl.ds(row0 + j * CH, CH)])
            pltpu.sync_copy(Hb_a, out_sh.at[pl.ds(row0 + j * CH, CH)])
        plsc.subcore_barrier()

        def issue(t, sbuf, dbuf, S, D, Hb, sem):
            @plsc.parallel_loop(0, CH // 16, unroll=CH // 16)
            def _ub(j):
                p = pbig[t, pl.ds(j * 16, 16)]
                sbuf[pl.ds(j * 16, 16)] = p & 0xFFFF
                dbuf[pl.ds(j * 16, 16)] = p >> 16

            pltpu.async_copy(as_hbm.at[sbuf], S, sem)
            pltpu.async_copy(ad_hbm.at[dbuf], D, sem)
            pltpu.async_copy(ht_hbm.at[sbuf], Hb, sem)

        def compute(sbuf, dbuf, S, D, V, Hb, sem):
            pltpu.make_async_copy(as_hbm.at[sbuf], S, sem).wait()
            pltpu.make_async_copy(ad_hbm.at[dbuf], D, sem).wait()
            pltpu.make_async_copy(ht_hbm.at[sbuf], Hb, sem).wait()

            @plsc.parallel_loop(0, CH, unroll=4)
            def _edge(i):
                e = S[i] + D[i]
                e = jnp.where(e >= 0.0, e, 0.2 * e)
                val = jnp.exp(e - k)
                V[i] = val
                for h in range(heads):
                    sp = _splat(val, h)
                    for q in range((width // heads) // 16):
                        col = h * (width // heads) + q * 16
                        Hb[i, pl.ds(col, 16)] = Hb[i, pl.ds(col, 16)] * sp
            cpv = pltpu.async_copy(V, den_sh.at[dbuf], sem, add=True)
            cph = pltpu.async_copy(Hb, out_sh.at[dbuf], sem, add=True)
            cpv.wait()
            cph.wait()

        issue(0, sbuf_a, dbuf_a, S_a, D_a, Hb_a, sem_a)

        def pair_body(tt, _):
            t0 = 2 * tt
            issue(t0 + 1, sbuf_b, dbuf_b, S_b, D_b, Hb_b, sem_b)
            compute(sbuf_a, dbuf_a, S_a, D_a, V_a, Hb_a, sem_a)

            @pl.when(tt < nch // 2 - 1)
            def _():
                issue(t0 + 2, sbuf_a, dbuf_a, S_a, D_a, Hb_a, sem_a)

            compute(sbuf_b, dbuf_b, S_b, D_b, V_b, Hb_b, sem_b)
            return _

        lax.fori_loop(0, nch // 2, pair_body, None)
        plsc.subcore_barrier()
        pltpu.sync_copy(den_sh.at[pl.ds(row0, RPT)],
                        dens_hbm.at[pl.ds(c * NPAD + row0, RPT)])
        pltpu.sync_copy(out_sh.at[pl.ds(row0, RPT)],
                        outs_hbm.at[pl.ds(c * NPAD + row0, RPT)])

    return edge


def kernel(inputs, edge_index, edge_attr, W1, a_src1, a_dst1, b1,
           W2, a_src2, a_dst2, b2):
    del edge_attr
    E = edge_index.shape[1]
    etot = E + N
    ept = ((etot + NW * 2 * CH - 1) // (NW * 2 * CH)) * 2 * CH
    epad = ept * NW

    loop = jnp.arange(N, dtype=jnp.int32)
    npad_e = epad - etot
    trash = jnp.int32(N) + (jnp.arange(npad_e, dtype=jnp.int32) % 16)
    src_p = jnp.concatenate([edge_index[0], loop, trash])
    dst_p = jnp.concatenate([edge_index[1], loop, trash])
    packed = (dst_p * jnp.int32(65536) + src_p).reshape(-1, CH)

    x = jnp.pad(inputs, ((0, NPAD - N), (0, 0)))

    eye1 = jnp.eye(H1, dtype=jnp.float32)
    aws1 = jnp.pad((a_src1[:, :, None] * eye1[:, None, :]).reshape(HC, H1),
                   ((0, 0), (0, 16 - H1)))
    awd1 = jnp.pad((a_dst1[:, :, None] * eye1[:, None, :]).reshape(HC, H1),
                   ((0, 0), (0, 16 - H1)))
    aws2 = jnp.pad(a_src2.T, ((0, 0), (0, 15)))
    awd2 = jnp.pad(a_dst2.T, ((0, 0), (0, 15)))

    lanes = jnp.arange(16)
    exp1 = (lanes[:, None] == (jnp.arange(HC) // (HC // H1))[None, :])
    exp1 = exp1.astype(jnp.float32)
    exp2 = (lanes[:, None] == jnp.zeros((NCLS,), jnp.int32)[None, :])
    exp2 = exp2.astype(jnp.float32)

    ht1, as1, ad1, mk1 = _make_prep1()(x, W1, aws1, awd1)
    dens1, outs1 = _make_edge(ept, H1, HC)(packed, as1, ad1, ht1, mk1)
    ht2, as2, ad2, mk2 = _make_prep2()(outs1, outs1, dens1, dens1, exp1,
                                       b1.reshape(1, HC), W2, aws2, awd2)
    dens2, outs2 = _make_edge(ept, 1, NCLS)(packed, as2, ad2, ht2, mk2)
    out = _make_final()(outs2, outs2, dens2, dens2, exp2,
                        b2.reshape(1, NCLS))
    return out[:N]
